# baseline (speedup 1.0000x reference)
import functools

import jax
import jax.numpy as jnp
from jax import lax
from jax.experimental import pallas as pl
from jax.experimental.pallas import tpu as pltpu
from jax.experimental.pallas import tpu_sc as plsc

N = 10000
NREL = 200
E = 320000
D = 128
H = 4
ALPHA = 0.1
NITER = 4

NC = 2
NS = 16
LANES = 16
CH = 128
NCHUNK = E // CH
MAXC_SC = NCHUNK // NS + 1
NPT = N // NS
NPC = 125
BM = 1000

_F32 = jnp.float32
_I32 = jnp.int32


def _tc_node_body(x_ref, g_ref, b_ref, wh_ref, wt_ref, ath_ref, att_ref,
                  z0_ref, az0_ref, ahat_ref):
    x = x_ref[0]
    mu = jnp.mean(x, axis=-1, keepdims=True)
    var = jnp.mean((x - mu) ** 2, axis=-1, keepdims=True)
    hn = (x - mu) / jnp.sqrt(var + 1e-5) * g_ref[...] + b_ref[...]
    z0_ref[0, 0] = hn[:, :64]
    z0_ref[1, 0] = hn[:, 64:]
    az0_ref[0, 0] = ALPHA * hn[:, :64]
    az0_ref[1, 0] = ALPHA * hn[:, 64:]
    dn = (((1,), (1,)), ((), ()))
    th = jnp.tanh(lax.dot_general(hn, wh_ref[...], dn, preferred_element_type=_F32))
    tt = jnp.tanh(lax.dot_general(hn, wt_ref[...], dn, preferred_element_type=_F32))
    ah = jnp.sum(th.reshape(NPT, H, D) * ath_ref[...].reshape(1, H, D), axis=-1)
    at = jnp.sum(tt.reshape(NPT, H, D) * att_ref[...].reshape(1, H, D), axis=-1)
    ahat_ref[0] = jnp.concatenate([ah, at], axis=1)


def _tc_node(entity_r, gamma, beta, wh, wt, ath, att):
    return pl.pallas_call(
        _tc_node_body,
        grid=(NS,),
        in_specs=[
            pl.BlockSpec((1, NPT, D), lambda i: (i, 0, 0)),
            pl.BlockSpec((1, D), lambda i: (0, 0)),
            pl.BlockSpec((1, D), lambda i: (0, 0)),
            pl.BlockSpec((H * D, D), lambda i: (0, 0)),
            pl.BlockSpec((H * D, D), lambda i: (0, 0)),
            pl.BlockSpec((H, D), lambda i: (0, 0)),
            pl.BlockSpec((H, D), lambda i: (0, 0)),
        ],
        out_specs=[
            pl.BlockSpec((NC, 1, NPT, 64), lambda i: (0, i, 0, 0)),
            pl.BlockSpec((NC, 1, NPT, 64), lambda i: (0, i, 0, 0)),
            pl.BlockSpec((1, NPT, 2 * H), lambda i: (i, 0, 0)),
        ],
        out_shape=[
            jax.ShapeDtypeStruct((NC, NS, NPT, 64), _F32),
            jax.ShapeDtypeStruct((NC, NS, NPT, 64), _F32),
            jax.ShapeDtypeStruct((NS, NPT, 2 * H), _F32),
        ],
    )(entity_r, gamma.reshape(1, D), beta.reshape(1, D), wh, wt,
      ath.reshape(H, D), att.reshape(H, D))


def _tc_rel_body(r_ref, wr_ref, atr_ref, ar_ref):
    dn = (((1,), (1,)), ((), ()))
    tr = jnp.tanh(lax.dot_general(r_ref[...], wr_ref[...], dn,
                                  preferred_element_type=_F32))
    ar_ref[...] = jnp.sum(tr.reshape(NREL, H, D) * atr_ref[...].reshape(1, H, D),
                          axis=-1)


def _tc_rel(relation, wr, atr):
    return pl.pallas_call(
        _tc_rel_body,
        out_shape=jax.ShapeDtypeStruct((NREL, H), _F32),
    )(relation, wr, atr.reshape(H, D))


def _tc_den_body(d_ref, o_ref):
    o_ref[0] = d_ref[0, 0, :, 0:H] + d_ref[1, 0, :, 0:H]


def _tc_den(denoms_r):
    return pl.pallas_call(
        _tc_den_body,
        grid=(NS,),
        in_specs=[pl.BlockSpec((NC, 1, NPT, 16), lambda i: (0, i, 0, 0))],
        out_specs=pl.BlockSpec((1, NPT, H), lambda i: (i, 0, 0)),
        out_shape=jax.ShapeDtypeStruct((NS, NPT, H), _F32),
    )(denoms_r)


def _tc_out_body(z_ref, wo_ref, x_ref, o_ref):
    acc = x_ref[0]
    dn = (((1,), (1,)), ((), ()))
    for h in range(H):
        for cc in range(NC):
            zb = z_ref[h, cc, 0]
            w = wo_ref[:, h * D + cc * 64:h * D + (cc + 1) * 64]
            acc = acc + lax.dot_general(zb, w, dn, preferred_element_type=_F32)
    o_ref[0] = acc


def _tc_out(zout, wo, x_r):
    return pl.pallas_call(
        _tc_out_body,
        grid=(NS,),
        in_specs=[
            pl.BlockSpec((H, NC, 1, NPT, 64), lambda i: (0, 0, i, 0, 0)),
            pl.BlockSpec((D, H * D), lambda i: (0, 0)),
            pl.BlockSpec((1, NPT, D), lambda i: (i, 0, 0)),
        ],
        out_specs=pl.BlockSpec((1, NPT, D), lambda i: (i, 0, 0)),
        out_shape=jax.ShapeDtypeStruct((NS, NPT, D), _F32),
    )(zout, wo, x_r)


_MESH = dict(core_axis_name="c", subcore_axis_name="s")


def _full(v):
    return jnp.full((LANES,), v, _I32)


@functools.partial(
    pl.kernel,
    out_type=[
        jax.ShapeDtypeStruct((H * E,), _F32),
        jax.ShapeDtypeStruct((NC, N, 16), _F32),
    ],
    mesh=plsc.VectorSubcoreMesh(**_MESH),
    compiler_params=pltpu.CompilerParams(needs_layout_passes=False, use_tc_tiling_on_sc=False),
    scratch_types=[
        pltpu.VMEM((N, 2 * H), _F32),
        pltpu.VMEM((NREL, H), _F32),
        pltpu.VMEM((CH,), _I32),
        pltpu.VMEM((CH,), _I32),
        pltpu.VMEM((CH,), _I32),
        pltpu.VMEM((CH, 16), _F32),
        pltpu.VMEM((H, CH), _F32),
        pltpu.VMEM_SHARED((N, 16), _F32),
    ],
)
def _s1(ahat_hbm, ar_hbm, eh_hbm, et_hbm, ety_hbm, exh_hbm, den_hbm,
        aht, arv, headb, tailb, typeb, ex16, exst, den_sp):
    c = lax.axis_index("c")
    s = lax.axis_index("s")
    w = c * NS + s
    pltpu.sync_copy(ahat_hbm, aht)
    pltpu.sync_copy(ar_hbm, arv)

    def _zro(i, carry):
        ex16[i, :] = jnp.zeros((LANES,), _F32)
        return carry
    lax.fori_loop(0, CH, _zro, 0)
    for j in range(NPT // NPC):
        pltpu.sync_copy(ex16.at[pl.ds(0, NPC)],
                        den_sp.at[pl.ds(s * NPT + j * NPC, NPC)])
    plsc.subcore_barrier()

    lo = (w * NCHUNK) // (NC * NS)
    hi = ((w + 1) * NCHUNK) // (NC * NS)

    def _chunk(k, carry):
        off = k * CH
        pltpu.sync_copy(eh_hbm.at[pl.ds(off, CH)], headb)
        pltpu.sync_copy(et_hbm.at[pl.ds(off, CH)], tailb)
        pltpu.sync_copy(ety_hbm.at[pl.ds(off, CH)], typeb)
        for h in range(H):
            for j in range(CH // LANES):
                sl = pl.ds(j * LANES, LANES)
                hv = headb[sl]
                tv = tailb[sl]
                rv = typeb[sl]
                a1 = plsc.load_gather(aht, [hv, _full(h)])
                a2 = plsc.load_gather(aht, [tv, _full(H + h)])
                a3 = plsc.load_gather(arv, [rv, _full(h)])
                sc = a1 + a2 + a3
                sc = jnp.where(sc > 0, sc, 0.01 * sc)
                ex = jnp.exp(sc)
                exst[h, sl] = ex
                rows = lax.iota(_I32, LANES) + j * LANES
                plsc.store_scatter(ex16, [rows, _full(h)], ex)
        for h in range(H):
            pltpu.sync_copy(exst.at[h], exh_hbm.at[pl.ds(h * E + off, CH)])
        pltpu.sync_copy(ex16, den_sp.at[headb], add=True)
        return carry
    lax.fori_loop(lo, hi, _chunk, 0)
    plsc.subcore_barrier()

    @pl.when(s == 0)
    def _():
        pltpu.sync_copy(den_sp, den_hbm.at[c])


@functools.partial(
    pl.kernel,
    out_type=jax.ShapeDtypeStruct((H * E,), _F32),
    mesh=plsc.VectorSubcoreMesh(**_MESH),
    compiler_params=pltpu.CompilerParams(needs_layout_passes=False, use_tc_tiling_on_sc=False),
    scratch_types=[
        pltpu.VMEM((N, H), _F32),
        pltpu.VMEM((CH,), _I32),
        pltpu.VMEM((H, CH), _F32),
        pltpu.VMEM((H, CH), _F32),
    ],
)
def _s2(exh_hbm, den4_hbm, eh_hbm, a_hbm, den4, headb, exb, ast):
    c = lax.axis_index("c")
    s = lax.axis_index("s")
    w = c * NS + s
    pltpu.sync_copy(den4_hbm, den4)
    lo = (w * NCHUNK) // (NC * NS)
    hi = ((w + 1) * NCHUNK) // (NC * NS)

    def _chunk(k, carry):
        off = k * CH
        pltpu.sync_copy(eh_hbm.at[pl.ds(off, CH)], headb)
        for h in range(H):
            pltpu.sync_copy(exh_hbm.at[pl.ds(h * E + off, CH)], exb.at[h])
        for h in range(H):
            for j in range(CH // LANES):
                sl = pl.ds(j * LANES, LANES)
                hv = headb[sl]
                dv = plsc.load_gather(den4, [hv, _full(h)])
                ast[h, sl] = (1.0 - ALPHA) * exb[h, sl] / (dv + 1e-30)
        for h in range(H):
            pltpu.sync_copy(ast.at[h], a_hbm.at[pl.ds(h * E + off, CH)])
        return carry
    lax.fori_loop(lo, hi, _chunk, 0)


NB = 5
KPT = 160
NBODY = KPT // (2 * NB)


@functools.partial(
    pl.kernel,
    out_type=jax.ShapeDtypeStruct((H, NC, NS, NPT, 64), _F32),
    mesh=plsc.VectorSubcoreMesh(**_MESH),
    compiler_params=pltpu.CompilerParams(needs_layout_passes=False, use_tc_tiling_on_sc=False),
    scratch_types=(
        [pltpu.VMEM_SHARED((N, 64), _F32)] * 2
        + [pltpu.VMEM((CH,), _I32)] * (2 * NB)
        + [pltpu.VMEM((CH,), _I32)] * (2 * NB)
        + [pltpu.VMEM((CH,), _F32)] * (2 * NB)
        + [pltpu.VMEM((CH, 64), _F32)] * NB
        + [pltpu.SemaphoreType.DMA] * (2 * NB)
        + [pltpu.SemaphoreType.DMA] * NB
        + [pltpu.SemaphoreType.DMA] * NB
    ),
)
def _s3(a_hbm, z0_hbm, az0_hbm, et_hbm, eh_hbm, zout_hbm, *refs):
    za_sp, zb_sp = refs[0], refs[1]
    o = 2
    tail = [refs[o:o + NB], refs[o + NB:o + 2 * NB]]
    o += 2 * NB
    head = [refs[o:o + NB], refs[o + NB:o + 2 * NB]]
    o += 2 * NB
    ac = [refs[o:o + NB], refs[o + NB:o + 2 * NB]]
    o += 2 * NB
    rows = refs[o:o + NB]
    o += NB
    esem = [refs[o:o + NB], refs[o + NB:o + 2 * NB]]
    o += 2 * NB
    gsem = refs[o:o + NB]
    o += NB
    ssem = refs[o:o + NB]
    c = lax.axis_index("c")
    s = lax.axis_index("s")
    klo = s * KPT
    kend = (s + 1) * KPT
    nb = s * NPT

    def _mul(rows_i, ac_i):
        def _m(r):
            wv = plsc.load_gather(ac_i, [_full(0) + r])
            for q in range(4):
                sl = pl.ds(q * LANES, LANES)
                rows_i[r, sl] = rows_i[r, sl] * wv
        plsc.parallel_loop(0, CH, 1, unroll=8)(_m)

    def _issue_edge(h, p, m, k):
        off = jnp.minimum(k, NCHUNK - 1) * CH
        pltpu.async_copy(et_hbm.at[pl.ds(off, CH)], tail[p][m], esem[p][m])
        pltpu.async_copy(eh_hbm.at[pl.ds(off, CH)], head[p][m], esem[p][m])
        pltpu.async_copy(a_hbm.at[pl.ds(h * E + off, CH)], ac[p][m], esem[p][m])

    def _wait_edge(p, m):
        pltpu.make_async_copy(et_hbm.at[pl.ds(0, CH)], tail[p][m], esem[p][m]).wait()
        pltpu.make_async_copy(eh_hbm.at[pl.ds(0, CH)], head[p][m], esem[p][m]).wait()
        pltpu.make_async_copy(a_hbm.at[pl.ds(0, CH)], ac[p][m], esem[p][m]).wait()

    def _wait_scat(nxt, p, m):
        pltpu.make_async_copy(rows[m], nxt.at[head[p][m]], ssem[m]).wait()

    def _half(h, cur, nxt, p, base, first):
        gd = []
        for m in range(NB):
            _wait_edge(p, m)
            if first is None:
                _wait_scat(nxt, 1 - p, m)
            else:
                @pl.when(first > 0)
                def _():
                    _wait_scat(nxt, 1 - p, m)

            @pl.when(base + NB + m < kend)
            def _():
                _issue_edge(h, 1 - p, m, base + NB + m)
            gd.append(pltpu.async_copy(cur.at[tail[p][m]], rows[m], gsem[m]))
        for m in range(NB):
            gd[m].wait()
            k = base + m

            @pl.when(k >= NCHUNK)
            def _():
                for q in range(CH // LANES):
                    ac[p][m][pl.ds(q * LANES, LANES)] = jnp.zeros((LANES,), _F32)
            _mul(rows[m], ac[p][m])
            pltpu.async_copy(rows[m], nxt.at[head[p][m]], ssem[m], add=True)

    def _head(h, carry):
        pltpu.sync_copy(z0_hbm.at[c, s], za_sp.at[pl.ds(nb, NPT)])
        cur = za_sp
        for it in range(NITER):
            nxt = (zb_sp, za_sp)[it % 2]
            pltpu.sync_copy(az0_hbm.at[c, s], nxt.at[pl.ds(nb, NPT)])
            plsc.subcore_barrier()

            for m in range(NB):
                _issue_edge(h, 0, m, klo + m)

            def _body(j, carry2):
                base = klo + j * 2 * NB
                _half(h, cur, nxt, 0, base, j)
                _half(h, cur, nxt, 1, base + NB, None)
                return carry2
            lax.fori_loop(0, NBODY, _body, 0)
            for m in range(NB):
                _wait_scat(nxt, 1, m)
            plsc.subcore_barrier()
            cur = nxt
        pltpu.sync_copy(cur.at[pl.ds(nb, NPT)], zout_hbm.at[h, c, s])
        return carry
    lax.fori_loop(0, H, _head, 0)


def kernel(params, edge_index, edge_type):
    entity = params['entity_embed']
    relation = params['relation_embed']
    e_head = edge_index[0]
    e_tail = edge_index[1]
    for lp in params['layers']:
        z0, az0, ahat = _tc_node(entity.reshape(NS, NPT, D),
                                 lp['norm_gamma'], lp['norm_beta'],
                                 lp['W_h'], lp['W_t'], lp['att_h'], lp['att_t'])
        ar = _tc_rel(relation, lp['W_r'], lp['att_r'])
        exh, denoms = _s1(ahat.reshape(N, 2 * H), ar, e_head, e_tail, edge_type)
        den4 = _tc_den(denoms.reshape(NC, NS, NPT, 16))
        a = _s2(exh, den4.reshape(N, H), e_head)
        zout = _s3(a, z0, az0, e_tail, e_head)
        entity = _tc_out(zout, lp['W_o'],
                         entity.reshape(NS, NPT, D)).reshape(N, D)
    return entity
